# baseline (speedup 1.0000x reference)
import functools

import jax
import jax.numpy as jnp
from jax import lax
from jax.experimental import pallas as pl
from jax.experimental.pallas import tpu as pltpu
from jax.experimental.pallas import tpu_sc as plsc

_B = 4096
_N = 4096
_K = 2048
_L = 16
_NW = 32
_ROUNDS = _N // (_NW * _L)


def _sc_body(x_hbm, o_hbm, chunk, hist):
    wid = lax.axis_index("c") * 16 + lax.axis_index("s")
    lanes = lax.iota(jnp.int32, 16)
    ones = jnp.ones((_L,), jnp.int32)
    zeros = jnp.zeros((_L,), jnp.int32)
    sign = jnp.full((_L,), -0x80000000, jnp.int32)
    m7f = jnp.full((_L,), 0x7FFFFFFF, jnp.int32)

    def zero_hist(b, _):
        hist[b, :] = zeros
        return 0

    def scan_hist(krem):
        def sbody(j, car):
            s, bsel, above, crossed = car
            b = 255 - j
            hv = hist[b, :]
            hist[b, :] = zeros
            s2 = s + hv
            cross_now = (s2 >= krem) & (crossed == 0)
            bsel = jnp.where(cross_now, b, bsel)
            above = jnp.where(cross_now, s, above)
            crossed = jnp.where(cross_now, ones, crossed)
            return (s2, bsel, above, crossed)
        init = (zeros, zeros, zeros, zeros)
        _, bsel, above, _ = lax.fori_loop(0, 256, sbody, init)
        return bsel, krem - above

    for r in range(_ROUNDS):
        base = (wid * _ROUNDS + r) * _L
        pltpu.sync_copy(x_hbm.at[:, pl.ds(base, _L)], chunk)
        lax.fori_loop(0, 256, zero_hist, 0)

        def p0(i, _):
            iv = plsc.bitcast(chunk[i, :], jnp.int32)
            flip = lax.shift_right_arithmetic(iv, 31) & m7f
            zv = (iv ^ flip) ^ sign
            chunk[i, :] = plsc.bitcast(zv, jnp.float32)
            b0 = lax.shift_right_logical(zv, 24)
            plsc.addupdate_scatter(hist, [b0, lanes], ones)
            return 0
        lax.fori_loop(0, _B, p0, 0)
        krem = jnp.full((_L,), _K, jnp.int32)
        prefix, krem = scan_hist(krem)

        for sm, sb in ((24, 16), (16, 8), (8, 0)):
            def pj(i, _, sm=sm, sb=sb, prefix=prefix):
                zv = plsc.bitcast(chunk[i, :], jnp.int32)
                match = lax.shift_right_logical(zv, sm) == prefix
                bj = lax.shift_right_logical(zv, sb) & 0xFF
                plsc.addupdate_scatter(hist, [bj, lanes], ones, mask=match)
                return 0
            lax.fori_loop(0, _B, pj, 0)
            bsel, krem = scan_hist(krem)
            prefix = lax.shift_left(prefix, 8) | bsel

        ty = prefix ^ sign
        def pm(i, _, ty=ty):
            zv = plsc.bitcast(chunk[i, :], jnp.int32)
            yv = zv ^ sign
            keep = yv >= ty
            flip = lax.shift_right_arithmetic(yv, 31) & m7f
            xv = plsc.bitcast(yv ^ flip, jnp.float32)
            chunk[i, :] = jnp.where(keep, xv, jnp.float32(0.0))
            return 0
        lax.fori_loop(0, _B, pm, 0)
        pltpu.sync_copy(chunk, o_hbm.at[:, pl.ds(base, _L)])


def kernel(x):
    mesh = plsc.VectorSubcoreMesh(core_axis_name="c", subcore_axis_name="s")
    kern = pl.kernel(
        _sc_body,
        out_type=jax.ShapeDtypeStruct((_B, _N), jnp.float32),
        mesh=mesh,
        scratch_types=[
            pltpu.VMEM((_B, _L), jnp.float32),
            pltpu.VMEM((256, _L), jnp.int32),
        ],
        compiler_params=pltpu.CompilerParams(use_tc_tiling_on_sc=False,
                                             needs_layout_passes=False),
    )
    return kern(x)


# speedup vs baseline: 3.6587x; 3.6587x over previous
import functools

import jax
import jax.numpy as jnp
from jax import lax
from jax.experimental import pallas as pl
from jax.experimental.pallas import tpu as pltpu
from jax.experimental.pallas import tpu_sc as plsc

_B = 4096
_N = 4096
_K = 2048
_L = 16
_NW = 32
_ROUNDS = _N // (_NW * _L)


def _sc_body(x_hbm, o_hbm, chunk, hist):
    wid = lax.axis_index("c") * 16 + lax.axis_index("s")
    lanes = lax.iota(jnp.int32, 16)
    ones = jnp.ones((_L,), jnp.int32)
    zeros = jnp.zeros((_L,), jnp.int32)
    sign = jnp.full((_L,), -0x80000000, jnp.int32)
    m7f = jnp.full((_L,), 0x7FFFFFFF, jnp.int32)

    @plsc.parallel_loop(0, 256, unroll=8)
    def _(b):
        hist[b, :] = zeros

    def scan_hist(krem):
        def sbody(j, car):
            s, bsel, above, crossed = car
            b = 255 - j
            hv = hist[b, :]
            hist[b, :] = zeros
            s2 = s + hv
            cross_now = (s2 >= krem) & (crossed == 0)
            bsel = jnp.where(cross_now, b, bsel)
            above = jnp.where(cross_now, s, above)
            crossed = jnp.where(cross_now, ones, crossed)
            return (s2, bsel, above, crossed)
        init = (zeros, zeros, zeros, zeros)
        _, bsel, above, _ = lax.fori_loop(0, 256, sbody, init)
        return bsel, krem - above

    for r in range(_ROUNDS):
        base = (wid * _ROUNDS + r) * _L
        pltpu.sync_copy(x_hbm.at[:, pl.ds(base, _L)], chunk)

        @plsc.parallel_loop(0, _B, unroll=16)
        def _(i):
            iv = plsc.bitcast(chunk[i, :], jnp.int32)
            flip = lax.shift_right_arithmetic(iv, 31) & m7f
            zv = (iv ^ flip) ^ sign
            chunk[i, :] = plsc.bitcast(zv, jnp.float32)
            b0 = lax.shift_right_logical(zv, 24)
            plsc.addupdate_scatter(hist, [b0, lanes], ones)

        krem = jnp.full((_L,), _K, jnp.int32)
        prefix, krem = scan_hist(krem)

        for sm, sb in ((24, 16), (16, 8), (8, 0)):
            @plsc.parallel_loop(0, _B, unroll=16)
            def _(i, sm=sm, sb=sb, prefix=prefix):
                zv = plsc.bitcast(chunk[i, :], jnp.int32)
                match = lax.shift_right_logical(zv, sm) == prefix
                bj = lax.shift_right_logical(zv, sb) & 0xFF
                plsc.addupdate_scatter(hist, [bj, lanes], ones, mask=match)
            bsel, krem = scan_hist(krem)
            prefix = lax.shift_left(prefix, 8) | bsel

        ty = prefix ^ sign

        @plsc.parallel_loop(0, _B, unroll=16)
        def _(i, ty=ty):
            zv = plsc.bitcast(chunk[i, :], jnp.int32)
            yv = zv ^ sign
            keep = yv >= ty
            flip = lax.shift_right_arithmetic(yv, 31) & m7f
            xv = plsc.bitcast(yv ^ flip, jnp.float32)
            chunk[i, :] = jnp.where(keep, xv, jnp.float32(0.0))

        pltpu.sync_copy(chunk, o_hbm.at[:, pl.ds(base, _L)])


def kernel(x):
    mesh = plsc.VectorSubcoreMesh(core_axis_name="c", subcore_axis_name="s")
    kern = pl.kernel(
        _sc_body,
        out_type=jax.ShapeDtypeStruct((_B, _N), jnp.float32),
        mesh=mesh,
        scratch_types=[
            pltpu.VMEM((_B, _L), jnp.float32),
            pltpu.VMEM((256, _L), jnp.int32),
        ],
        compiler_params=pltpu.CompilerParams(use_tc_tiling_on_sc=False,
                                             needs_layout_passes=False),
    )
    return kern(x)


# speedup vs baseline: 3.8510x; 1.0526x over previous
import functools

import jax
import jax.numpy as jnp
from jax import lax
from jax.experimental import pallas as pl
from jax.experimental.pallas import tpu as pltpu
from jax.experimental.pallas import tpu_sc as plsc

_B = 4096
_N = 4096
_K = 2048
_L = 16
_NW = 32
_ROUNDS = _N // (_NW * _L)


def _sc_body(x_hbm, o_hbm, chunk, hist):
    wid = lax.axis_index("c") * 16 + lax.axis_index("s")
    lanes = lax.iota(jnp.int32, 16)
    ones = jnp.ones((_L,), jnp.int32)
    zeros = jnp.zeros((_L,), jnp.int32)
    sign = jnp.full((_L,), -0x80000000, jnp.int32)
    m7f = jnp.full((_L,), 0x7FFFFFFF, jnp.int32)

    @plsc.parallel_loop(0, 256, unroll=8)
    def _(b):
        hist[b, :] = zeros

    def scan_hist(krem):
        def sbody(j, car):
            s, bsel, above, crossed = car
            b = 255 - j
            hv = hist[b, :]
            hist[b, :] = zeros
            s2 = s + hv
            cross_now = (s2 >= krem) & (crossed == 0)
            bsel = jnp.where(cross_now, b, bsel)
            above = jnp.where(cross_now, s, above)
            crossed = jnp.where(cross_now, ones, crossed)
            return (s2, bsel, above, crossed)
        init = (zeros, zeros, zeros, zeros)
        _, bsel, above, _ = plsc.parallel_loop(0, 256, unroll=8,
                                               carry=init)(sbody)
        return bsel, krem - above

    for r in range(_ROUNDS):
        base = (wid * _ROUNDS + r) * _L
        pltpu.sync_copy(x_hbm.at[:, pl.ds(base, _L)], chunk)

        @plsc.parallel_loop(0, _B, unroll=16)
        def _(i):
            iv = plsc.bitcast(chunk[i, :], jnp.int32)
            zv = iv ^ (lax.shift_right_arithmetic(iv, 31) | sign)
            chunk[i, :] = plsc.bitcast(zv, jnp.float32)
            b0 = lax.shift_right_logical(zv, 24)
            plsc.addupdate_scatter(hist, [b0, lanes], ones)

        krem = jnp.full((_L,), _K, jnp.int32)
        prefix, krem = scan_hist(krem)

        for sm, sb in ((24, 16), (16, 8), (8, 0)):
            @plsc.parallel_loop(0, _B, unroll=16)
            def _(i, sm=sm, sb=sb, prefix=prefix):
                zv = plsc.bitcast(chunk[i, :], jnp.int32)
                match = lax.shift_right_logical(zv, sm) == prefix
                bj = lax.shift_right_logical(zv, sb) & 0xFF
                plsc.addupdate_scatter(hist, [bj, lanes], ones, mask=match)
            bsel, krem = scan_hist(krem)
            prefix = lax.shift_left(prefix, 8) | bsel

        ty = prefix ^ sign

        @plsc.parallel_loop(0, _B, unroll=16)
        def _(i, ty=ty):
            zv = plsc.bitcast(chunk[i, :], jnp.int32)
            yv = zv ^ sign
            keep = yv >= ty
            flip = lax.shift_right_arithmetic(yv, 31) & m7f
            xv = plsc.bitcast(yv ^ flip, jnp.float32)
            chunk[i, :] = jnp.where(keep, xv, jnp.float32(0.0))

        pltpu.sync_copy(chunk, o_hbm.at[:, pl.ds(base, _L)])


def kernel(x):
    mesh = plsc.VectorSubcoreMesh(core_axis_name="c", subcore_axis_name="s")
    kern = pl.kernel(
        _sc_body,
        out_type=jax.ShapeDtypeStruct((_B, _N), jnp.float32),
        mesh=mesh,
        scratch_types=[
            pltpu.VMEM((_B, _L), jnp.float32),
            pltpu.VMEM((256, _L), jnp.int32),
        ],
        compiler_params=pltpu.CompilerParams(use_tc_tiling_on_sc=False,
                                             needs_layout_passes=False),
    )
    return kern(x)
